# Initial kernel scaffold; baseline (speedup 1.0000x reference)
import jax
import jax.numpy as jnp
from jax.experimental import pallas as pl


def kernel(images, locs):
    raise NotImplementedError("write your pallas kernel here")


# speedup vs baseline: 37.4058x; 37.4058x over previous
import jax
import jax.numpy as jnp
from jax.experimental import pallas as pl
from jax.experimental.pallas import tpu as pltpu

_B, _C, _S, _OUT = 64, 3, 448, 224


def _racnn_body(locs_ref, img_ref, out_ref):
    b = pl.program_id(0)
    fS = jnp.float32(_S)
    tx = locs_ref[b, 0]
    ty = locs_ref[b, 1]
    tl = locs_ref[b, 2]
    tl = jnp.clip(tl, fS / 3.0, fS * 2.0 / 3.0)
    tx = jnp.clip(tx, tl, fS - tl)
    ty = jnp.clip(ty, tl, fS - tl)
    w_off = jnp.maximum(jnp.floor(tx - tl), 0.0)
    w_end = jnp.where(tx + tl < fS, jnp.floor(tx + tl), fS)
    h_off = jnp.maximum(jnp.floor(ty - tl), 0.0)
    h_end = jnp.where(ty + tl < fS, jnp.floor(ty + tl), fS)

    def weights(off, end, shape, out_axis):
        r = jax.lax.broadcasted_iota(jnp.int32, shape, out_axis).astype(jnp.float32)
        s = jax.lax.broadcasted_iota(jnp.int32, shape, 1 - out_axis).astype(jnp.float32)
        L = end - off
        src = off + r * (L - 1.0) / (_OUT - 1.0)
        i0 = jnp.clip(jnp.floor(src), 0.0, fS - 1.0)
        i1 = jnp.minimum(i0 + 1.0, fS - 1.0)
        fr = src - i0
        w = jnp.where(s == i0, 1.0 - fr, 0.0) + jnp.where(s == i1, fr, 0.0)
        mask = jax.nn.sigmoid(10.0 * (s - off)) - jax.nn.sigmoid(10.0 * (s - end))
        return w * mask

    a_w = weights(w_off, w_end, (_OUT, _S), 0)
    b_w = weights(h_off, h_end, (_S, _OUT), 1)

    img = img_ref[0]
    y = jnp.dot(img, b_w, preferred_element_type=jnp.float32)
    for c in range(_C):
        out_ref[0, c] = jnp.dot(a_w, y[c * _S:(c + 1) * _S, :],
                                preferred_element_type=jnp.float32)


def kernel(images, locs):
    imgs2 = images.reshape(_B, _C * _S, _S)
    return pl.pallas_call(
        _racnn_body,
        grid=(_B,),
        in_specs=[
            pl.BlockSpec(memory_space=pltpu.SMEM),
            pl.BlockSpec((1, _C * _S, _S), lambda b: (b, 0, 0)),
        ],
        out_specs=pl.BlockSpec((1, _C, _OUT, _OUT), lambda b: (b, 0, 0, 0)),
        out_shape=jax.ShapeDtypeStruct((_B, _C, _OUT, _OUT), jnp.float32),
        compiler_params=pltpu.CompilerParams(
            dimension_semantics=("parallel",),
        ),
    )(locs, imgs2)
